# Initial kernel scaffold; baseline (speedup 1.0000x reference)
import jax
import jax.numpy as jnp
from jax.experimental import pallas as pl


def kernel(coeffs, embedding):
    raise NotImplementedError("write your pallas kernel here")


# speedup vs baseline: 4.3307x; 4.3307x over previous
import functools

import jax
import jax.numpy as jnp
from jax import lax
from jax.experimental import pallas as pl
from jax.experimental.pallas import tpu as pltpu
from jax.experimental.pallas import tpu_sc as plsc

_NUM_CORES = 2
_NUM_SUBCORES = 16
_NW = _NUM_CORES * _NUM_SUBCORES
_G = 128


@functools.cache
def _build(n_rows: int, n_groups: int, p: int, d: int):
    mesh = plsc.VectorSubcoreMesh(
        core_axis_name="c", subcore_axis_name="s",
        num_cores=_NUM_CORES, num_subcores=_NUM_SUBCORES,
    )

    @functools.partial(
        pl.kernel,
        out_type=jax.ShapeDtypeStruct((n_rows, d), jnp.float32),
        mesh=mesh,
        scratch_types=[
            pltpu.VMEM((n_groups, _G), jnp.int32),
            pltpu.VMEM((_G, d), jnp.float32),
            pltpu.SemaphoreType.DMA,
        ],
        compiler_params=pltpu.CompilerParams(use_tc_tiling_on_sc=False),
    )
    def kern(idx_hbm, table_hbm, out_hbm, idx_v, rows_v, sem):
        wid = lax.axis_index("s") * _NUM_CORES + lax.axis_index("c")
        base = wid * (n_groups * _G)
        pltpu.sync_copy(idx_hbm.at[wid], idx_v)

        def body(g, carry):
            pltpu.async_copy(table_hbm.at[idx_v.at[g]], rows_v, sem).wait()
            pltpu.sync_copy(rows_v, out_hbm.at[pl.ds(base + g * _G, _G)])
            return carry

        lax.fori_loop(0, n_groups, body, 0)

    return kern


def kernel(coeffs, embedding):
    batch, p_dim = coeffs.shape
    p, d = embedding.shape
    n = batch * p_dim
    chunk = _NW * _G
    n_pad = -(-n // chunk) * chunk
    idx = coeffs.reshape(-1).astype(jnp.int32)
    if n_pad != n:
        idx = jnp.pad(idx, (0, n_pad - n))
    n_groups = n_pad // chunk
    idx = idx.reshape(_NW, n_groups, _G)
    out = _build(n_pad, n_groups, p, d)(idx, embedding)
    if n_pad != n:
        out = out[:n]
    return out.reshape(batch, p_dim, d)


# speedup vs baseline: 5.3652x; 1.2389x over previous
import functools

import jax
import jax.numpy as jnp
from jax import lax
from jax.experimental import pallas as pl
from jax.experimental.pallas import tpu as pltpu
from jax.experimental.pallas import tpu_sc as plsc

_NUM_CORES = 2
_NUM_SUBCORES = 16
_NW = _NUM_CORES * _NUM_SUBCORES
_G = 128


def _pick_pipeline(n_groups: int):
    for k, nb in ((5, 2), (2, 5), (5, 5), (2, 2), (1, 2), (1, 5)):
        if n_groups % k:
            continue
        n_super = n_groups // k
        if n_super % nb == 0 and n_super >= 2 * nb:
            return k, nb, min(nb - 1, 1 if nb == 2 else 3)
    return None


@functools.cache
def _build(n_rows: int, n_groups: int, d: int):
    mesh = plsc.VectorSubcoreMesh(
        core_axis_name="c", subcore_axis_name="s",
        num_cores=_NUM_CORES, num_subcores=_NUM_SUBCORES,
    )
    cfg = _pick_pipeline(n_groups)

    if cfg is None:
        @functools.partial(
            pl.kernel,
            out_type=jax.ShapeDtypeStruct((n_rows, d), jnp.float32),
            mesh=mesh,
            scratch_types=[
                pltpu.VMEM((n_groups, _G), jnp.int32),
                pltpu.VMEM((_G, d), jnp.float32),
                pltpu.SemaphoreType.DMA,
            ],
            compiler_params=pltpu.CompilerParams(use_tc_tiling_on_sc=False),
        )
        def kern_serial(idx_hbm, table_hbm, out_hbm, idx_v, rows_v, sem):
            wid = lax.axis_index("s") * _NUM_CORES + lax.axis_index("c")
            base = wid * (n_groups * _G)

            def body(g, carry):
                pltpu.async_copy(table_hbm.at[idx_v.at[g]], rows_v, sem).wait()
                pltpu.sync_copy(rows_v, out_hbm.at[pl.ds(base + g * _G, _G)])
                return carry

            lax.fori_loop(0, n_groups, body, 0)

        return kern_serial

    k_g, nb, f = cfg
    n_super = n_groups // k_g
    chunk_rows = k_g * _G
    n_steady = (n_super - nb) // nb

    @functools.partial(
        pl.kernel,
        out_type=jax.ShapeDtypeStruct((n_rows, d), jnp.float32),
        mesh=mesh,
        scratch_types=(
            [pltpu.VMEM((n_groups, _G), jnp.int32)]
            + [pltpu.VMEM((chunk_rows, d), jnp.float32) for _ in range(nb)]
            + [pltpu.SemaphoreType.DMA for _ in range(2 * nb)]
        ),
        compiler_params=pltpu.CompilerParams(use_tc_tiling_on_sc=False),
    )
    def kern(idx_hbm, table_hbm, out_hbm, idx_v, *bufs_sems):
        rows = bufs_sems[:nb]
        gsem = bufs_sems[nb:2 * nb]
        ssem = bufs_sems[2 * nb:]
        wid = lax.axis_index("s") * _NUM_CORES + lax.axis_index("c")
        base = wid * (n_groups * _G)
        pltpu.sync_copy(idx_hbm.at[wid], idx_v)

        def fire(t, b):
            for j in range(k_g):
                pltpu.async_copy(
                    table_hbm.at[idx_v.at[t * k_g + j]],
                    rows[b].at[pl.ds(j * _G, _G)],
                    gsem[b],
                )

        def drain(b):
            for j in range(k_g):
                pltpu.make_async_copy(
                    table_hbm.at[idx_v.at[0]],
                    rows[b].at[pl.ds(j * _G, _G)],
                    gsem[b],
                ).wait()

        def scatter(t, b):
            pltpu.async_copy(
                rows[b], out_hbm.at[pl.ds(base + t * chunk_rows, chunk_rows)],
                ssem[b],
            )

        def wait_scatter(b):
            pltpu.make_async_copy(
                rows[b], out_hbm.at[pl.ds(base, chunk_rows)], ssem[b],
            ).wait()

        for t in range(f):
            fire(t, t)
        for t in range(nb - f):
            fire(t + f, (t + f) % nb)
            drain(t % nb)
            scatter(t, t % nb)

        def body(i, carry):
            for off in range(nb):
                t = (nb - f) + i * nb + off
                b = (nb - f + off) % nb
                bf = (nb + off) % nb
                wait_scatter(bf)
                fire(t + f, bf)
                drain(b)
                scatter(t, b)
            return carry

        if n_steady:
            lax.fori_loop(0, n_steady, body, 0)
        for j in range(f):
            t = n_super - f + j
            drain(t % nb)
            scatter(t, t % nb)
        for b in range(nb):
            wait_scatter(b)

    return kern


def kernel(coeffs, embedding):
    batch, p_dim = coeffs.shape
    _, d = embedding.shape
    n = batch * p_dim
    chunk = _NW * _G
    n_pad = -(-n // chunk) * chunk
    idx = coeffs.reshape(-1).astype(jnp.int32)
    if n_pad != n:
        idx = jnp.pad(idx, (0, n_pad - n))
    n_groups = n_pad // chunk
    idx = idx.reshape(_NW, n_groups, _G)
    out = _build(n_pad, n_groups, d)(idx, embedding)
    if n_pad != n:
        out = out[:n]
    return out.reshape(batch, p_dim, d)


# speedup vs baseline: 8.2818x; 1.5436x over previous
import functools

import jax
import jax.numpy as jnp
from jax import lax
from jax.experimental import pallas as pl
from jax.experimental.pallas import tpu as pltpu
from jax.experimental.pallas import tpu_sc as plsc

_NUM_CORES = 2
_NUM_SUBCORES = 16
_NW = _NUM_CORES * _NUM_SUBCORES
_L = 16


@functools.cache
def _build(n_rows: int, p: int, d: int, rows_w: int, chunk: int):
    n_chunks = rows_w // chunk
    grps = chunk // _L
    mesh = plsc.VectorSubcoreMesh(
        core_axis_name="c", subcore_axis_name="s",
        num_cores=_NUM_CORES, num_subcores=_NUM_SUBCORES,
    )

    @functools.partial(
        pl.kernel,
        out_type=jax.ShapeDtypeStruct((n_rows, d), jnp.float32),
        mesh=mesh,
        scratch_types=[
            pltpu.VMEM((rows_w,), jnp.int32),
            pltpu.VMEM((p * d,), jnp.float32),
            pltpu.VMEM((chunk, d), jnp.float32),
            pltpu.VMEM((chunk, d), jnp.float32),
            pltpu.SemaphoreType.DMA,
            pltpu.SemaphoreType.DMA,
        ],
        compiler_params=pltpu.CompilerParams(use_tc_tiling_on_sc=True),
    )
    def kern(idx_hbm, table_hbm, out_hbm, idx_v, table_v, b0, b1, s0, s1):
        wid = lax.axis_index("s") * _NUM_CORES + lax.axis_index("c")
        base = wid * rows_w
        pltpu.sync_copy(idx_hbm.at[pl.ds(base, rows_w)], idx_v)
        pltpu.sync_copy(table_hbm, table_v)

        def fill(c, b):
            def grp(j, carry):
                iv = idx_v[pl.ds(c * chunk + j * _L, _L)] * d
                for l in range(_L):
                    b[j * _L + l, :] = table_v[pl.ds(iv[l], d)]
                return carry

            lax.fori_loop(0, grps, grp, 0)

        def flush(c, b, s):
            pltpu.async_copy(b, out_hbm.at[pl.ds(base + c * chunk, chunk)], s)

        def wait(b, s):
            pltpu.make_async_copy(b, out_hbm.at[pl.ds(base, chunk)], s).wait()

        fill(0, b0)
        flush(0, b0, s0)

        def body(i, carry):
            c = 2 * i
            fill(c + 1, b1)
            flush(c + 1, b1, s1)
            wait(b0, s0)
            fill(c + 2, b0)
            flush(c + 2, b0, s0)
            wait(b1, s1)
            return carry

        lax.fori_loop(0, (n_chunks - 1) // 2, body, 0)
        if n_chunks % 2 == 0:
            fill(n_chunks - 1, b1)
            flush(n_chunks - 1, b1, s1)
            wait(b1, s1)
        wait(b0, s0)

    return kern


def kernel(coeffs, embedding):
    batch, p_dim = coeffs.shape
    p, d = embedding.shape
    n = batch * p_dim
    chunk = 256
    quantum = _NW * chunk
    n_pad = -(-n // quantum) * quantum
    idx = coeffs.reshape(-1).astype(jnp.int32)
    if n_pad != n:
        idx = jnp.pad(idx, (0, n_pad - n))
    rows_w = n_pad // _NW
    out = _build(n_pad, p, d, rows_w, chunk)(idx, embedding.reshape(-1))
    if n_pad != n:
        out = out[:n]
    return out.reshape(batch, p_dim, d)
